# baseline (speedup 1.0000x reference)
import jax
import jax.numpy as jnp
from jax import lax
from jax.experimental import pallas as pl
from jax.experimental.pallas import tpu as pltpu
from jax.experimental.pallas import tpu_sc as plsc

D = 16
CH = 32


def _sc_geometry():
    try:
        info = plsc.get_sparse_core_info()
        return info.num_cores, info.num_subcores
    except Exception:
        return 2, 16


def _tr_body(tin, tout):
    x = tin[...]
    L = x.shape[1]
    y = jnp.transpose(x)
    y3 = y.reshape(L // 8, 8, D)
    parts = [y3[:, s, :] for s in range(8)]
    tout[...] = jnp.concatenate(parts, axis=1)


def _tc_transpose(tabT, V):
    L = 8192
    grid = (V + L - 1) // L
    return pl.pallas_call(
        _tr_body,
        grid=(grid,),
        in_specs=[pl.BlockSpec((16, L), lambda i: (0, i))],
        out_specs=pl.BlockSpec((L // 8, 128), lambda i: (i, 0)),
        out_shape=jax.ShapeDtypeStruct((V // 8, 128), jnp.float32),
    )(tabT)


def _sc_gather(user_idx, item_idx, tab_u, tab_i, B):
    NC, NS = _sc_geometry()
    NW = NC * NS
    rpw = B // NW
    n_ch = rpw // CH

    mesh = plsc.VectorSubcoreMesh(core_axis_name="c", subcore_axis_name="s")

    def body(uidx_hbm, iidx_hbm, tabu_hbm, tabi_hbm,
             out_u, out_ib, out_i,
             idxu_v, idxi_v, divu_v, divi_v, pb, ob, sem):
        wid = lax.axis_index("s") * NC + lax.axis_index("c")
        base = wid * rpw
        pltpu.sync_copy(uidx_hbm.at[pl.ds(base, rpw)], idxu_v)
        pltpu.sync_copy(iidx_hbm.at[pl.ds(base, rpw)], idxi_v)
        for g in range(rpw // D):
            sl = pl.ds(g * D, D)
            divu_v[sl] = lax.shift_right_logical(idxu_v[sl], 3)
            divi_v[sl] = lax.shift_right_logical(idxi_v[sl], 3)

        def run_stream(idx_v, div_v, tab, out):
            def wave(j, _):
                pltpu.async_copy(tab.at[div_v.at[pl.ds(j * CH, CH)]], pb,
                                 sem).wait()
                for g in range(CH // D):
                    vec = idx_v[pl.ds(j * CH + g * D, D)]
                    for l in range(D):
                        r = vec[l] & 7
                        off = pl.multiple_of(r * D, D)
                        ob[g * D + l, :] = pb[g * D + l, pl.ds(off, D)]
                pltpu.sync_copy(ob, out.at[pl.ds(base + j * CH, CH)])
                return 0

            lax.fori_loop(0, n_ch, wave, 0)

        run_stream(idxu_v, divu_v, tabu_hbm, out_u)
        run_stream(idxi_v, divi_v, tabu_hbm, out_ib)
        run_stream(idxi_v, divi_v, tabi_hbm, out_i)

    out_sds = jax.ShapeDtypeStruct((B, D), jnp.float32)
    k = pl.kernel(
        body,
        out_type=(out_sds, out_sds, out_sds),
        mesh=mesh,
        scratch_types=[
            pltpu.VMEM((rpw,), jnp.int32),
            pltpu.VMEM((rpw,), jnp.int32),
            pltpu.VMEM((rpw,), jnp.int32),
            pltpu.VMEM((rpw,), jnp.int32),
            pltpu.VMEM((CH, 128), jnp.float32),
            pltpu.VMEM((CH, D), jnp.float32),
            pltpu.SemaphoreType.DMA,
        ],
    )
    return k(user_idx, item_idx, tab_u, tab_i)


def _mlp_body(ru, rib, ri, w1t, b1, w2t, b2, w3t, b3, womf, womlp, bo, out):
    u = ru[...]
    x = jnp.concatenate([u, ri[...]], axis=1)
    hp = jax.lax.Precision.HIGHEST
    h = jnp.maximum(jnp.dot(x, w1t[...], precision=hp) + b1[...], 0.0)
    h = jnp.maximum(jnp.dot(h, w2t[...], precision=hp) + b2[...], 0.0)
    h = jnp.maximum(jnp.dot(h, w3t[...], precision=hp) + b3[...], 0.0)
    mf = u * rib[...]
    logit = (jnp.dot(mf, womf[...], precision=hp)
             + jnp.dot(h, womlp[...], precision=hp) + bo[...])
    out[...] = jax.nn.sigmoid(logit)


def _tc_mlp(ru, rib, ri, W1, b1, W2, b2, W3, b3, Wo, bo, B):
    blk = 4096
    grid = B // blk
    full = lambda shape: pl.BlockSpec(shape, lambda i: (0, 0))
    row = lambda: pl.BlockSpec((blk, D), lambda i: (i, 0))
    return pl.pallas_call(
        _mlp_body,
        grid=(grid,),
        in_specs=[
            row(), row(), row(),
            full((32, 32)), full((1, 32)),
            full((32, 16)), full((1, 16)),
            full((16, 8)), full((1, 8)),
            full((16, 1)), full((8, 1)), full((1, 1)),
        ],
        out_specs=pl.BlockSpec((blk, 1), lambda i: (i, 0)),
        out_shape=jax.ShapeDtypeStruct((B, 1), jnp.float32),
    )(ru, rib, ri,
      W1.T, b1.reshape(1, 32),
      W2.T, b2.reshape(1, 16),
      W3.T, b3.reshape(1, 8),
      Wo[:, :D].T, Wo[:, D:].T, bo.reshape(1, 1))


def kernel(user_input, item_input, mf_user_table, mf_item_table,
           W1, b1, W2, b2, W3, b3, Wo, bo):
    B = user_input.shape[0]
    V = mf_user_table.shape[0]
    tab_u = _tc_transpose(mf_user_table.T, V)
    tab_i = _tc_transpose(mf_item_table.T, V)
    ru, rib, ri = _sc_gather(user_input, item_input, tab_u, tab_i, B)
    return _tc_mlp(ru, rib, ri, W1, b1, W2, b2, W3, b3, Wo, bo, B)


# speedup vs baseline: 1.0660x; 1.0660x over previous
import jax
import jax.numpy as jnp
from jax import lax
from jax.experimental import pallas as pl
from jax.experimental.pallas import tpu as pltpu
from jax.experimental.pallas import tpu_sc as plsc

D = 16
CH = 32


def _sc_geometry():
    try:
        info = plsc.get_sparse_core_info()
        return info.num_cores, info.num_subcores
    except Exception:
        return 2, 16


def _sc_gather(user_idx, item_idx, tab_u, tab_i, B):
    NC, NS = _sc_geometry()
    NW = NC * NS
    rpw = B // NW
    n_ch = rpw // CH

    mesh = plsc.VectorSubcoreMesh(core_axis_name="c", subcore_axis_name="s")

    def body(uidx_hbm, iidx_hbm, tabu_hbm, tabi_hbm,
             out_u, out_ib, out_i,
             idxu_v, idxi_v, pb, ob, sem):
        wid = lax.axis_index("s") * NC + lax.axis_index("c")
        base = wid * rpw
        pltpu.sync_copy(uidx_hbm.at[pl.ds(base, rpw)], idxu_v)
        pltpu.sync_copy(iidx_hbm.at[pl.ds(base, rpw)], idxi_v)

        def run_stream(idx_v, tab, out):
            def wave(j, _):
                vecs = [idx_v[pl.ds(j * CH + g * D, D)] for g in range(CH // D)]
                for g, vec in enumerate(vecs):
                    for l in range(D):
                        v = vec[l]
                        row8 = pl.multiple_of(
                            lax.shift_left(lax.shift_right_logical(v, 3), 3), 8)
                        slot = (g * D + l) * 8
                        pltpu.async_copy(tab.at[pl.ds(row8, 8), :],
                                         pb.at[pl.ds(slot, 8), :], sem)
                pltpu.make_async_copy(tab.at[pl.ds(0, CH * 8), :], pb,
                                      sem).wait()
                for g, vec in enumerate(vecs):
                    for l in range(D):
                        r = vec[l] & 7
                        val = pb[(g * D + l) * 8 + r, :]
                        ob[g * D + l, :] = val
                pltpu.sync_copy(ob, out.at[pl.ds(base + j * CH, CH)])
                return 0

            lax.fori_loop(0, n_ch, wave, 0)

        run_stream(idxu_v, tabu_hbm, out_u)
        run_stream(idxi_v, tabu_hbm, out_ib)
        run_stream(idxi_v, tabi_hbm, out_i)

    out_sds = jax.ShapeDtypeStruct((B, D), jnp.float32)
    k = pl.kernel(
        body,
        out_type=(out_sds, out_sds, out_sds),
        mesh=mesh,
        scratch_types=[
            pltpu.VMEM((rpw,), jnp.int32),
            pltpu.VMEM((rpw,), jnp.int32),
            pltpu.VMEM((CH * 8, D), jnp.float32),
            pltpu.VMEM((CH, D), jnp.float32),
            pltpu.SemaphoreType.DMA,
        ],
    )
    return k(user_idx, item_idx, tab_u, tab_i)


def _mlp_body(ru, rib, ri, w1t, b1, w2t, b2, w3t, b3, womf, womlp, bo, out):
    u = ru[...]
    x = jnp.concatenate([u, ri[...]], axis=1)
    hp = jax.lax.Precision.HIGHEST
    h = jnp.maximum(jnp.dot(x, w1t[...], precision=hp) + b1[...], 0.0)
    h = jnp.maximum(jnp.dot(h, w2t[...], precision=hp) + b2[...], 0.0)
    h = jnp.maximum(jnp.dot(h, w3t[...], precision=hp) + b3[...], 0.0)
    mf = u * rib[...]
    logit = (jnp.dot(mf, womf[...], precision=hp)
             + jnp.dot(h, womlp[...], precision=hp) + bo[...])
    out[...] = jax.nn.sigmoid(logit)


def _tc_mlp(ru, rib, ri, W1, b1, W2, b2, W3, b3, Wo, bo, B):
    blk = 4096
    grid = B // blk
    full = lambda shape: pl.BlockSpec(shape, lambda i: (0, 0))
    row = lambda: pl.BlockSpec((blk, D), lambda i: (i, 0))
    return pl.pallas_call(
        _mlp_body,
        grid=(grid,),
        in_specs=[
            row(), row(), row(),
            full((32, 32)), full((1, 32)),
            full((32, 16)), full((1, 16)),
            full((16, 8)), full((1, 8)),
            full((16, 1)), full((8, 1)), full((1, 1)),
        ],
        out_specs=pl.BlockSpec((blk, 1), lambda i: (i, 0)),
        out_shape=jax.ShapeDtypeStruct((B, 1), jnp.float32),
    )(ru, rib, ri,
      W1.T, b1.reshape(1, 32),
      W2.T, b2.reshape(1, 16),
      W3.T, b3.reshape(1, 8),
      Wo[:, :D].T, Wo[:, D:].T, bo.reshape(1, 1))


def kernel(user_input, item_input, mf_user_table, mf_item_table,
           W1, b1, W2, b2, W3, b3, Wo, bo):
    B = user_input.shape[0]
    ru, rib, ri = _sc_gather(user_input, item_input,
                             mf_user_table, mf_item_table, B)
    return _tc_mlp(ru, rib, ri, W1, b1, W2, b2, W3, b3, Wo, bo, B)
